# Initial kernel scaffold; baseline (speedup 1.0000x reference)
import functools

import jax
import jax.numpy as jnp
from jax import lax
from jax.experimental import pallas as pl
from jax.experimental.pallas import tpu as pltpu
from jax.experimental.pallas import tpu_sc as plsc

N = 10000
E = 320000
D = 128
G = 64

NC, NS, LANES = 2, 16, 16
NW = NC * NS
NPAD = 10240
CHUNK = 128
CPT = 79
EPAD = NW * CPT * CHUNK
ROWS_PER_TILE = NPAD // NS
BLK = 1024
NBLK = NPAD // BLK

_MESH = plsc.VectorSubcoreMesh(core_axis_name="c", subcore_axis_name="s")


def _sc_segsum(hs, src, dst, zeros):
    @functools.partial(
        pl.kernel,
        out_type=jax.ShapeDtypeStruct((NC, NPAD, D), jnp.float32),
        mesh=_MESH,
        scratch_types=[
            pltpu.VMEM((CHUNK,), jnp.int32),
            pltpu.VMEM((CHUNK,), jnp.int32),
            pltpu.VMEM((CHUNK, D), jnp.float32),
            pltpu.VMEM_SHARED((NPAD, D), jnp.float32),
            pltpu.SemaphoreType.DMA,
        ],
    )
    def k(hs_hbm, src_hbm, dst_hbm, z_hbm, out_hbm, sidx, didx, rows, acc, sem):
        cid = lax.axis_index("c")
        sid = lax.axis_index("s")
        wid = cid * NS + sid
        r0 = sid * ROWS_PER_TILE
        pltpu.sync_copy(z_hbm.at[pl.ds(r0, ROWS_PER_TILE)],
                        acc.at[pl.ds(r0, ROWS_PER_TILE)])
        plsc.subcore_barrier()
        base = wid * (CPT * CHUNK)

        @pl.loop(0, CPT)
        def _(ch):
            off = base + ch * CHUNK
            pltpu.sync_copy(src_hbm.at[pl.ds(off, CHUNK)], sidx)
            pltpu.sync_copy(dst_hbm.at[pl.ds(off, CHUNK)], didx)
            pltpu.async_copy(hs_hbm.at[sidx], rows, sem).wait()
            pltpu.sync_copy(rows, acc.at[didx], add=True)

        plsc.subcore_barrier()
        pltpu.sync_copy(acc.at[pl.ds(r0, ROWS_PER_TILE)],
                        out_hbm.at[cid, pl.ds(r0, ROWS_PER_TILE)])

    return k(hs, src, dst, zeros)


def _sc_deg(dst, zeros16):
    @functools.partial(
        pl.kernel,
        out_type=jax.ShapeDtypeStruct((NC, NPAD, LANES), jnp.float32),
        mesh=_MESH,
        scratch_types=[
            pltpu.VMEM((CHUNK,), jnp.int32),
            pltpu.VMEM((CHUNK, LANES), jnp.float32),
            pltpu.VMEM_SHARED((NPAD, LANES), jnp.float32),
        ],
    )
    def k(dst_hbm, z_hbm, out_hbm, didx, ones, acc):
        cid = lax.axis_index("c")
        sid = lax.axis_index("s")
        wid = cid * NS + sid
        r0 = sid * ROWS_PER_TILE

        @pl.loop(0, CHUNK)
        def _(i):
            ones[i, :] = jnp.full((LANES,), 1.0, jnp.float32)

        pltpu.sync_copy(z_hbm.at[pl.ds(r0, ROWS_PER_TILE)],
                        acc.at[pl.ds(r0, ROWS_PER_TILE)])
        plsc.subcore_barrier()
        base = wid * (CPT * CHUNK)

        @pl.loop(0, CPT)
        def _(ch):
            off = base + ch * CHUNK
            pltpu.sync_copy(dst_hbm.at[pl.ds(off, CHUNK)], didx)
            pltpu.sync_copy(ones, acc.at[didx], add=True)

        plsc.subcore_barrier()
        pltpu.sync_copy(acc.at[pl.ds(r0, ROWS_PER_TILE)],
                        out_hbm.at[cid, pl.ds(r0, ROWS_PER_TILE)])

    return k(dst, zeros16)


def _dinv_of(dp_ref):
    deg = dp_ref[0, :, 0:1] + dp_ref[1, :, 0:1] + 1.0
    return lax.rsqrt(deg)


def _tc_matmul(x, W):
    def body(x_ref, w_ref, o_ref):
        o_ref[...] = jnp.dot(x_ref[...], w_ref[...],
                             preferred_element_type=jnp.float32)

    return pl.pallas_call(
        body,
        grid=(NBLK,),
        in_specs=[pl.BlockSpec((BLK, D), lambda i: (i, 0)),
                  pl.BlockSpec((D, D), lambda i: (0, 0))],
        out_specs=pl.BlockSpec((BLK, D), lambda i: (i, 0)),
        out_shape=jax.ShapeDtypeStruct((NPAD, D), jnp.float32),
    )(x, W)


def _tc_scale(h, degP):
    def body(h_ref, dp_ref, o_ref):
        o_ref[...] = h_ref[...] * _dinv_of(dp_ref)

    return pl.pallas_call(
        body,
        grid=(NBLK,),
        in_specs=[pl.BlockSpec((BLK, D), lambda i: (i, 0)),
                  pl.BlockSpec((NC, BLK, LANES), lambda i: (0, i, 0))],
        out_specs=pl.BlockSpec((BLK, D), lambda i: (i, 0)),
        out_shape=jax.ShapeDtypeStruct((NPAD, D), jnp.float32),
    )(h, degP)


def _tc_layer(S, hs, degP, b, W2):
    def body(s_ref, hs_ref, dp_ref, b_ref, w_ref, o_ref):
        dinv = _dinv_of(dp_ref)
        z = jnp.maximum((s_ref[0] + s_ref[1] + hs_ref[...]) * dinv + b_ref[...],
                        0.0)
        o_ref[...] = jnp.dot(z, w_ref[...],
                             preferred_element_type=jnp.float32) * dinv

    return pl.pallas_call(
        body,
        grid=(NBLK,),
        in_specs=[pl.BlockSpec((NC, BLK, D), lambda i: (0, i, 0)),
                  pl.BlockSpec((BLK, D), lambda i: (i, 0)),
                  pl.BlockSpec((NC, BLK, LANES), lambda i: (0, i, 0)),
                  pl.BlockSpec((1, D), lambda i: (0, 0)),
                  pl.BlockSpec((D, D), lambda i: (0, 0))],
        out_specs=pl.BlockSpec((BLK, D), lambda i: (i, 0)),
        out_shape=jax.ShapeDtypeStruct((NPAD, D), jnp.float32),
    )(S, hs, degP, b, W2)


def _tc_head(S, hs, degP, b, batch3, Wlp, blp):
    def body(s_ref, hs_ref, dp_ref, b_ref, bt_ref, wl_ref, bl_ref, o_ref,
             pooled, cnt):
        i = pl.program_id(0)

        @pl.when(i == 0)
        def _():
            pooled[...] = jnp.zeros_like(pooled)
            cnt[...] = jnp.zeros_like(cnt)

        dinv = _dinv_of(dp_ref)
        z = jnp.maximum((s_ref[0] + s_ref[1] + hs_ref[...]) * dinv + b_ref[...],
                        0.0)
        bt = bt_ref[0, 0, :]
        oh = (bt[:, None] ==
              lax.broadcasted_iota(jnp.int32, (1, G), 1)).astype(jnp.float32)
        dn = (((0,), (0,)), ((), ()))
        pooled[...] += lax.dot_general(oh, z, dn,
                                       preferred_element_type=jnp.float32)
        cnt[...] += lax.dot_general(oh, jnp.ones_like(z), dn,
                                    preferred_element_type=jnp.float32)

        @pl.when(i == NBLK - 1)
        def _():
            pm = pooled[...] / jnp.maximum(cnt[...], 1.0)
            o_ref[...] = jnp.dot(pm, wl_ref[...],
                                 preferred_element_type=jnp.float32) + bl_ref[...]

    return pl.pallas_call(
        body,
        grid=(NBLK,),
        in_specs=[pl.BlockSpec((NC, BLK, D), lambda i: (0, i, 0)),
                  pl.BlockSpec((BLK, D), lambda i: (i, 0)),
                  pl.BlockSpec((NC, BLK, LANES), lambda i: (0, i, 0)),
                  pl.BlockSpec((1, D), lambda i: (0, 0)),
                  pl.BlockSpec((1, 1, BLK), lambda i: (i, 0, 0)),
                  pl.BlockSpec((D, D), lambda i: (0, 0)),
                  pl.BlockSpec((1, D), lambda i: (0, 0))],
        out_specs=pl.BlockSpec((G, D), lambda i: (0, 0)),
        out_shape=jax.ShapeDtypeStruct((G, D), jnp.float32),
        scratch_shapes=[pltpu.VMEM((G, D), jnp.float32),
                        pltpu.VMEM((G, D), jnp.float32)],
    )(S, hs, degP, b, batch3, Wlp, blp)


def kernel(x, edge_index, batch, W1, b1, W2, b2, Wl, bl):
    f32 = jnp.float32
    pad_e = EPAD - E
    srcp = jnp.concatenate(
        [edge_index[0], jnp.full((pad_e,), NPAD - 1, jnp.int32)])
    dstp = jnp.concatenate(
        [edge_index[1], jnp.full((pad_e,), NPAD - 1, jnp.int32)])
    xp = jnp.pad(x, ((0, NPAD - N), (0, 0)))
    batchp = jnp.pad(batch, (0, NPAD - N), constant_values=G)
    batch3 = batchp.reshape(NBLK, 1, BLK)
    zeros = jnp.zeros((NPAD, D), f32)
    zeros16 = jnp.zeros((NPAD, LANES), f32)
    b1r = b1.reshape(1, D)
    b2r = b2.reshape(1, D)
    Wlp = jnp.pad(Wl, ((0, 0), (0, D - 1)))
    blp = jnp.pad(bl.reshape(1, 1), ((0, 0), (0, D - 1)))

    degP = _sc_deg(dstp, zeros16)
    h1 = _tc_matmul(xp, W1)
    hs1 = _tc_scale(h1, degP)
    S1 = _sc_segsum(hs1, srcp, dstp, zeros)
    hs2 = _tc_layer(S1, hs1, degP, b1r, W2)
    S2 = _sc_segsum(hs2, srcp, dstp, zeros)
    res = _tc_head(S2, hs2, degP, b2r, batch3, Wlp, blp)
    return res[:, :1]


# speedup vs baseline: 10.1011x; 10.1011x over previous
import functools

import jax
import jax.numpy as jnp
from jax import lax
from jax.experimental import pallas as pl
from jax.experimental.pallas import tpu as pltpu
from jax.experimental.pallas import tpu_sc as plsc

N = 10000
E = 320000
D = 128
G = 64

NC, NS, LANES = 2, 16, 16
NW = NC * NS
NPAD = 10240
CHUNK = 128
CPT = 79
EPAD = NW * CPT * CHUNK
ROWS_PER_TILE = NPAD // NS
BLK = 1024
NBLK = NPAD // BLK

def _mesh():
    return plsc.VectorSubcoreMesh(core_axis_name="c", subcore_axis_name="s")


def _sc_segsum(hs, src, dst, zeros):
    @functools.partial(
        pl.kernel,
        out_type=jax.ShapeDtypeStruct((NC, NPAD, D), jnp.float32),
        mesh=_mesh(),
        scratch_types=[
            pltpu.VMEM((CHUNK,), jnp.int32),
            pltpu.VMEM((CHUNK,), jnp.int32),
            pltpu.VMEM((CHUNK, D), jnp.float32),
            pltpu.VMEM_SHARED((NPAD, D), jnp.float32),
            pltpu.SemaphoreType.DMA,
        ],
    )
    def k(hs_hbm, src_hbm, dst_hbm, z_hbm, out_hbm, sidx, didx, rows, acc, sem):
        cid = lax.axis_index("c")
        sid = lax.axis_index("s")
        wid = cid * NS + sid
        r0 = sid * ROWS_PER_TILE
        pltpu.sync_copy(z_hbm.at[pl.ds(r0, ROWS_PER_TILE)],
                        acc.at[pl.ds(r0, ROWS_PER_TILE)])
        plsc.subcore_barrier()
        base = wid * (CPT * CHUNK)

        @pl.loop(0, CPT)
        def _(ch):
            off = base + ch * CHUNK
            pltpu.sync_copy(src_hbm.at[pl.ds(off, CHUNK)], sidx)
            pltpu.sync_copy(dst_hbm.at[pl.ds(off, CHUNK)], didx)
            pltpu.async_copy(hs_hbm.at[sidx], rows, sem).wait()
            pltpu.sync_copy(rows, acc.at[didx], add=True)

        plsc.subcore_barrier()
        pltpu.sync_copy(acc.at[pl.ds(r0, ROWS_PER_TILE)],
                        out_hbm.at[cid, pl.ds(r0, ROWS_PER_TILE)])

    return k(hs, src, dst, zeros)


def _sc_deg(dst, zeros16):
    @functools.partial(
        pl.kernel,
        out_type=jax.ShapeDtypeStruct((NC, NPAD, LANES), jnp.float32),
        mesh=_mesh(),
        scratch_types=[
            pltpu.VMEM((CHUNK,), jnp.int32),
            pltpu.VMEM((CHUNK, LANES), jnp.float32),
            pltpu.VMEM_SHARED((NPAD, LANES), jnp.float32),
        ],
    )
    def k(dst_hbm, z_hbm, out_hbm, didx, ones, acc):
        cid = lax.axis_index("c")
        sid = lax.axis_index("s")
        wid = cid * NS + sid
        r0 = sid * ROWS_PER_TILE

        @pl.loop(0, CHUNK)
        def _(i):
            ones[i, :] = jnp.full((LANES,), 1.0, jnp.float32)

        pltpu.sync_copy(z_hbm.at[pl.ds(r0, ROWS_PER_TILE)],
                        acc.at[pl.ds(r0, ROWS_PER_TILE)])
        plsc.subcore_barrier()
        base = wid * (CPT * CHUNK)

        @pl.loop(0, CPT)
        def _(ch):
            off = base + ch * CHUNK
            pltpu.sync_copy(dst_hbm.at[pl.ds(off, CHUNK)], didx)
            pltpu.sync_copy(ones, acc.at[didx], add=True)

        plsc.subcore_barrier()
        pltpu.sync_copy(acc.at[pl.ds(r0, ROWS_PER_TILE)],
                        out_hbm.at[cid, pl.ds(r0, ROWS_PER_TILE)])

    return k(dst, zeros16)


def _dinv_of(dp_ref):
    deg = dp_ref[0, :, 0:1] + dp_ref[1, :, 0:1] + 1.0
    return lax.rsqrt(deg)


def _tc_matmul(x, W):
    def body(x_ref, w_ref, o_ref):
        o_ref[...] = jnp.dot(x_ref[...], w_ref[...],
                             preferred_element_type=jnp.float32)

    return pl.pallas_call(
        body,
        grid=(NBLK,),
        in_specs=[pl.BlockSpec((BLK, D), lambda i: (i, 0)),
                  pl.BlockSpec((D, D), lambda i: (0, 0))],
        out_specs=pl.BlockSpec((BLK, D), lambda i: (i, 0)),
        out_shape=jax.ShapeDtypeStruct((NPAD, D), jnp.float32),
    )(x, W)


def _tc_scale(h, degP):
    def body(h_ref, dp_ref, o_ref):
        o_ref[...] = h_ref[...] * _dinv_of(dp_ref)

    return pl.pallas_call(
        body,
        grid=(NBLK,),
        in_specs=[pl.BlockSpec((BLK, D), lambda i: (i, 0)),
                  pl.BlockSpec((NC, BLK, LANES), lambda i: (0, i, 0))],
        out_specs=pl.BlockSpec((BLK, D), lambda i: (i, 0)),
        out_shape=jax.ShapeDtypeStruct((NPAD, D), jnp.float32),
    )(h, degP)


def _tc_layer(S, hs, degP, b, W2):
    def body(s_ref, hs_ref, dp_ref, b_ref, w_ref, o_ref):
        dinv = _dinv_of(dp_ref)
        z = jnp.maximum((s_ref[0] + s_ref[1] + hs_ref[...]) * dinv + b_ref[...],
                        0.0)
        o_ref[...] = jnp.dot(z, w_ref[...],
                             preferred_element_type=jnp.float32) * dinv

    return pl.pallas_call(
        body,
        grid=(NBLK,),
        in_specs=[pl.BlockSpec((NC, BLK, D), lambda i: (0, i, 0)),
                  pl.BlockSpec((BLK, D), lambda i: (i, 0)),
                  pl.BlockSpec((NC, BLK, LANES), lambda i: (0, i, 0)),
                  pl.BlockSpec((1, D), lambda i: (0, 0)),
                  pl.BlockSpec((D, D), lambda i: (0, 0))],
        out_specs=pl.BlockSpec((BLK, D), lambda i: (i, 0)),
        out_shape=jax.ShapeDtypeStruct((NPAD, D), jnp.float32),
    )(S, hs, degP, b, W2)


def _tc_head(S, hs, degP, b, batch3, Wlp, blp):
    def body(s_ref, hs_ref, dp_ref, b_ref, bt_ref, wl_ref, bl_ref, o_ref,
             pooled, cnt):
        i = pl.program_id(0)

        @pl.when(i == 0)
        def _():
            pooled[...] = jnp.zeros_like(pooled)
            cnt[...] = jnp.zeros_like(cnt)

        dinv = _dinv_of(dp_ref)
        z = jnp.maximum((s_ref[0] + s_ref[1] + hs_ref[...]) * dinv + b_ref[...],
                        0.0)
        bt = bt_ref[0, 0, :]
        oh = (bt[:, None] ==
              lax.broadcasted_iota(jnp.int32, (1, G), 1)).astype(jnp.float32)
        dn = (((0,), (0,)), ((), ()))
        pooled[...] += lax.dot_general(oh, z, dn,
                                       preferred_element_type=jnp.float32)
        cnt[...] += lax.dot_general(oh, jnp.ones_like(z), dn,
                                    preferred_element_type=jnp.float32)

        @pl.when(i == NBLK - 1)
        def _():
            pm = pooled[...] / jnp.maximum(cnt[...], 1.0)
            o_ref[...] = jnp.dot(pm, wl_ref[...],
                                 preferred_element_type=jnp.float32) + bl_ref[...]

    return pl.pallas_call(
        body,
        grid=(NBLK,),
        in_specs=[pl.BlockSpec((NC, BLK, D), lambda i: (0, i, 0)),
                  pl.BlockSpec((BLK, D), lambda i: (i, 0)),
                  pl.BlockSpec((NC, BLK, LANES), lambda i: (0, i, 0)),
                  pl.BlockSpec((1, D), lambda i: (0, 0)),
                  pl.BlockSpec((1, 1, BLK), lambda i: (i, 0, 0)),
                  pl.BlockSpec((D, D), lambda i: (0, 0)),
                  pl.BlockSpec((1, D), lambda i: (0, 0))],
        out_specs=pl.BlockSpec((G, D), lambda i: (0, 0)),
        out_shape=jax.ShapeDtypeStruct((G, D), jnp.float32),
        scratch_shapes=[pltpu.VMEM((G, D), jnp.float32),
                        pltpu.VMEM((G, D), jnp.float32)],
    )(S, hs, degP, b, batch3, Wlp, blp)


def kernel(x, edge_index, batch, W1, b1, W2, b2, Wl, bl):
    f32 = jnp.float32
    pad_e = EPAD - E
    srcp = jnp.concatenate(
        [edge_index[0], jnp.full((pad_e,), NPAD - 1, jnp.int32)])
    dstp = jnp.concatenate(
        [edge_index[1], jnp.full((pad_e,), NPAD - 1, jnp.int32)])
    xp = jnp.pad(x, ((0, NPAD - N), (0, 0)))
    batchp = jnp.pad(batch, (0, NPAD - N), constant_values=G)
    batch3 = batchp.reshape(NBLK, 1, BLK)
    zeros = jnp.zeros((NPAD, D), f32)
    zeros16 = jnp.zeros((NPAD, LANES), f32)
    b1r = b1.reshape(1, D)
    b2r = b2.reshape(1, D)
    Wlp = jnp.pad(Wl, ((0, 0), (0, D - 1)))
    blp = jnp.pad(bl.reshape(1, 1), ((0, 0), (0, D - 1)))

    degP = _sc_deg(dstp, zeros16)
    h1 = _tc_matmul(xp, W1)
    hs1 = _tc_scale(h1, degP)
    S1 = _sc_segsum(hs1, srcp, dstp, zeros)
    hs2 = _tc_layer(S1, hs1, degP, b1r, W2)
    S2 = _sc_segsum(hs2, srcp, dstp, zeros)
    res = _tc_head(S2, hs2, degP, b2r, batch3, Wlp, blp)
    return res[:, :1]
